# Initial kernel scaffold; baseline (speedup 1.0000x reference)
import jax
import jax.numpy as jnp
from jax.experimental import pallas as pl


def kernel(loc_b, loc_c, loc_z, loc_y, loc_x, x_os, y_os, z_os, ints, psf_volume, channel_facs):
    raise NotImplementedError("write your pallas kernel here")


# speedup vs baseline: 1059.6342x; 1059.6342x over previous
import functools

import jax
import jax.numpy as jnp
from jax import lax
from jax.experimental import pallas as pl
from jax.experimental.pallas import tpu as pltpu
from jax.experimental.pallas import tpu_sc as plsc

_N = 8192
_BS, _C, _D, _H, _W = 2, 2, 32, 512, 512
_SZ, _SY, _SX = 7, 15, 15
_SCALE = 10000.0

_NC = 2
_NS = 16
_L = 16

_ROUND_Z = 2
_NROUNDS = _D // _ROUND_Z
_SLAB_WORDS = _C * _ROUND_Z * _H * _W
_TEC_WB = _SLAB_WORDS // _NS
_DHW = _D * _H * _W
_HW = _H * _W
_CHUNK = _N // _NS
_NBUF_ROWS = 32
_DUMP = _SLAB_WORDS
_ZERO_W = 16384


def _body(lb, lc, lz, ly, lx, xo, yo, zo, it, pad, cf,
          out,
          acc,
          pad_v, lb_v, lc_v, lz_v, ly_v, lx_v,
          xo_v, yo_v, zo_v, it_v, cf_v,
          zero_v, val_v, idx_v):
    cid = lax.axis_index("c")
    sid = lax.axis_index("s")
    base_e = sid * _CHUNK

    def sload(ref, i):
        return ref[pl.ds(i, _L)][0]

    pltpu.sync_copy(pad, pad_v)
    pltpu.sync_copy(cf, cf_v)
    pltpu.sync_copy(lb.at[pl.ds(base_e, _CHUNK)], lb_v.at[pl.ds(0, _CHUNK)])
    pltpu.sync_copy(lc.at[pl.ds(base_e, _CHUNK)], lc_v.at[pl.ds(0, _CHUNK)])
    pltpu.sync_copy(lz.at[pl.ds(base_e, _CHUNK)], lz_v.at[pl.ds(0, _CHUNK)])
    pltpu.sync_copy(ly.at[pl.ds(base_e, _CHUNK)], ly_v.at[pl.ds(0, _CHUNK)])
    pltpu.sync_copy(lx.at[pl.ds(base_e, _CHUNK)], lx_v.at[pl.ds(0, _CHUNK)])
    pltpu.sync_copy(xo.at[pl.ds(base_e, _CHUNK)], xo_v.at[pl.ds(0, _CHUNK)])
    pltpu.sync_copy(yo.at[pl.ds(base_e, _CHUNK)], yo_v.at[pl.ds(0, _CHUNK)])
    pltpu.sync_copy(zo.at[pl.ds(base_e, _CHUNK)], zo_v.at[pl.ds(0, _CHUNK)])
    pltpu.sync_copy(it.at[pl.ds(base_e, _CHUNK)], it_v.at[pl.ds(0, _CHUNK)])

    ii = lax.iota(jnp.int32, _L)
    zvec = jnp.zeros((_L,), jnp.float32)

    def zb(j, _):
        zero_v[pl.ds(j * _L, _L)] = zvec
        return 0
    lax.fori_loop(0, _ZERO_W // _L, zb, 0)

    dump_idx = _DUMP + ii

    def round_body(r, _):
        z0 = r * _ROUND_Z

        def zr(j, _):
            pltpu.sync_copy(
                zero_v, acc.at[pl.ds(sid * _TEC_WB + j * _ZERO_W, _ZERO_W)])
            return 0
        lax.fori_loop(0, _TEC_WB // _ZERO_W, zr, 0)
        plsc.subcore_barrier()

        def emitter_body(i, rowcnt):
            elz = sload(lz_v, i)
            elb = sload(lb_v, i)
            zlo = jnp.maximum(z0, elz - (_SZ // 2))
            zhi = jnp.minimum(z0 + _ROUND_Z - 1, elz + (_SZ // 2))
            zub = jnp.where(elb == cid, zhi + 1, zlo)

            def z_body(zz, rowcnt):
                elc = sload(lc_v, i)
                ely = sload(ly_v, i)
                elx = sload(lx_v, i)
                dz = sload(zo_v, i) - 0.5
                dy = sload(yo_v, i) - 0.5
                dx = sload(xo_v, i) - 0.5
                fzi = jnp.where(dz < 0.0, -1, 0)
                fyi = jnp.where(dy < 0.0, -1, 0)
                fxi = jnp.where(dx < 0.0, -1, 0)
                wz1 = dz - fzi.astype(jnp.float32)
                wy1 = dy - fyi.astype(jnp.float32)
                wx1 = dx - fxi.astype(jnp.float32)
                wz0 = 1.0 - wz1
                wy0 = 1.0 - wy1
                wx0 = 1.0 - wx1
                cfv = cf_v[...]
                fac = sload(it_v, i) * _SCALE * jnp.where(elc == 0, cfv[0], cfv[1])
                t00 = wy0 * wx0 * fac
                t01 = wy0 * wx1 * fac
                t10 = wy1 * wx0 * fac
                t11 = wy1 * wx1 * fac
                c000 = wz0 * t00
                c001 = wz0 * t01
                c010 = wz0 * t10
                c011 = wz0 * t11
                c100 = wz1 * t00
                c101 = wz1 * t01
                c110 = wz1 * t10
                c111 = wz1 * t11

                k = zz - elz + (_SZ // 2)
                jz0 = k + 1 + fzi
                jz1 = jz0 + 1
                x0 = 1 + fxi
                x1 = x0 + 1
                sbase_z = (elc * _ROUND_Z + (zz - z0)) * _HW
                xbase = elx - (_SX // 2)
                xv = xbase + ii
                xok = (xv >= 0) & (xv < _W) & (ii < _SX)

                dylo = jnp.maximum(0, (_SY // 2) - ely)
                dyhi = jnp.minimum(_SY - 1, (_H - 1) - ely + (_SY // 2))

                def y_body(dyy, rowcnt):
                    jy0 = dyy + 1 + fyi
                    jy1 = jy0 + 1
                    a00 = pad_v[jz0, jy0, pl.ds(x0, _L)]
                    a00s = pad_v[jz0, jy0, pl.ds(x1, _L)]
                    a01 = pad_v[jz0, jy1, pl.ds(x0, _L)]
                    a01s = pad_v[jz0, jy1, pl.ds(x1, _L)]
                    a10 = pad_v[jz1, jy0, pl.ds(x0, _L)]
                    a10s = pad_v[jz1, jy0, pl.ds(x1, _L)]
                    a11 = pad_v[jz1, jy1, pl.ds(x0, _L)]
                    a11s = pad_v[jz1, jy1, pl.ds(x1, _L)]
                    row = (a00 * c000 + a00s * c001 +
                           a01 * c010 + a01s * c011 +
                           a10 * c100 + a10s * c101 +
                           a11 * c110 + a11s * c111)
                    y = ely - (_SY // 2) + dyy
                    widx = sbase_z + y * _W + xv
                    rowidx = jnp.where(xok, widx, dump_idx)
                    pos = rowcnt * _L
                    val_v[pl.ds(pos, _L)] = row
                    idx_v[pl.ds(pos, _L)] = rowidx
                    rowcnt = rowcnt + 1

                    @pl.when(rowcnt == _NBUF_ROWS)
                    def _():
                        pltpu.sync_copy(val_v, acc.at[idx_v], add=True)

                    return jnp.where(rowcnt == _NBUF_ROWS, 0, rowcnt)

                return lax.fori_loop(dylo, dyhi + 1, y_body, rowcnt)

            return lax.fori_loop(zlo, zub, z_body, rowcnt)

        rowcnt = lax.fori_loop(0, _CHUNK, emitter_body, 0)

        def pad_body(j, _):
            pos = j * _L
            val_v[pl.ds(pos, _L)] = zvec
            idx_v[pl.ds(pos, _L)] = dump_idx
            return 0
        lax.fori_loop(rowcnt, _NBUF_ROWS, pad_body, 0)
        pltpu.sync_copy(val_v, acc.at[idx_v], add=True)

        plsc.subcore_barrier()

        bcl = sid // 8
        zof = (sid // 4) % 2
        yq = sid % 4
        hbm_off = ((2 * cid + bcl) * _DHW + (z0 + zof) * _HW
                   + yq * (_H // 4) * _W)
        pltpu.sync_copy(acc.at[pl.ds(sid * _TEC_WB, _TEC_WB)],
                        out.at[pl.ds(hbm_off, _TEC_WB)])
        return 0

    lax.fori_loop(0, _NROUNDS, round_body, 0)


@jax.jit
def _sc_place(lb, lc, lz, ly, lx, xo, yo, zo, it, pad, cf):
    mesh = plsc.VectorSubcoreMesh(core_axis_name="c", subcore_axis_name="s",
                                  num_cores=_NC, num_subcores=_NS)
    f = pl.kernel(
        _body,
        out_type=jax.ShapeDtypeStruct((_BS * _C * _D * _H * _W,), jnp.float32),
        mesh=mesh,
        scratch_types=[
            pltpu.VMEM_SHARED((_SLAB_WORDS + 64,), jnp.float32),
            pltpu.VMEM((_SZ + 2, _SY + 2, 24), jnp.float32),
            pltpu.VMEM((_CHUNK + _L,), jnp.int32),
            pltpu.VMEM((_CHUNK + _L,), jnp.int32),
            pltpu.VMEM((_CHUNK + _L,), jnp.int32),
            pltpu.VMEM((_CHUNK + _L,), jnp.int32),
            pltpu.VMEM((_CHUNK + _L,), jnp.int32),
            pltpu.VMEM((_CHUNK + _L,), jnp.float32),
            pltpu.VMEM((_CHUNK + _L,), jnp.float32),
            pltpu.VMEM((_CHUNK + _L,), jnp.float32),
            pltpu.VMEM((_CHUNK + _L,), jnp.float32),
            pltpu.VMEM((_L,), jnp.float32),
            pltpu.VMEM((_ZERO_W,), jnp.float32),
            pltpu.VMEM((_NBUF_ROWS * _L,), jnp.float32),
            pltpu.VMEM((_NBUF_ROWS * _L,), jnp.int32),
        ],
    )
    return f(lb, lc, lz, ly, lx, xo, yo, zo, it, pad, cf)


def kernel(loc_b, loc_c, loc_z, loc_y, loc_x, x_os, y_os, z_os, ints,
           psf_volume, channel_facs):
    psfc = jnp.maximum(psf_volume.astype(jnp.float32), 0.0)
    pad = jnp.pad(psfc, ((1, 1), (1, 1), (1, 24 - _SX - 1)))
    cf = jnp.zeros((_L,), jnp.float32).at[:_C].set(
        channel_facs.astype(jnp.float32))
    out = _sc_place(
        loc_b.astype(jnp.int32), loc_c.astype(jnp.int32),
        loc_z.astype(jnp.int32), loc_y.astype(jnp.int32),
        loc_x.astype(jnp.int32),
        x_os.astype(jnp.float32), y_os.astype(jnp.float32),
        z_os.astype(jnp.float32), ints.astype(jnp.float32),
        pad, cf)
    return out.reshape(_BS, _C, _D, _H, _W)


# speedup vs baseline: 1558.5992x; 1.4709x over previous
import functools

import jax
import jax.numpy as jnp
from jax import lax
from jax.experimental import pallas as pl
from jax.experimental.pallas import tpu as pltpu
from jax.experimental.pallas import tpu_sc as plsc

_N = 8192
_BS, _C, _D, _H, _W = 2, 2, 32, 512, 512
_SZ, _SY, _SX = 7, 15, 15
_SCALE = 10000.0

_NC = 2
_NS = 16
_L = 16

_ROUND_Z = 2
_NROUNDS = _D // _ROUND_Z
_SLAB_WORDS = _C * _ROUND_Z * _H * _W
_TEC_WB = _SLAB_WORDS // _NS
_DHW = _D * _H * _W
_HW = _H * _W
_CHUNK = _N // _NS
_NBUF_ROWS = 60
_DUMP = _SLAB_WORDS
_ZERO_W = 16384
_ACC_EXTRA = 7424


def _body(lb, lc, lz, ly, lx, xo, yo, zo, it, pad, cf,
          out,
          acc,
          pad_v, lb_v, lc_v, lz_v, ly_v, lx_v,
          xo_v, yo_v, zo_v, it_v, cf_v,
          zero_v, val_v, idx_v):
    cid = lax.axis_index("c")
    sid = lax.axis_index("s")
    base_e = sid * _CHUNK

    def sload(ref, i):
        return ref[pl.ds(i, _L)][0]

    pltpu.sync_copy(pad, pad_v)
    pltpu.sync_copy(cf, cf_v)
    pltpu.sync_copy(lb.at[pl.ds(base_e, _CHUNK)], lb_v.at[pl.ds(0, _CHUNK)])
    pltpu.sync_copy(lc.at[pl.ds(base_e, _CHUNK)], lc_v.at[pl.ds(0, _CHUNK)])
    pltpu.sync_copy(lz.at[pl.ds(base_e, _CHUNK)], lz_v.at[pl.ds(0, _CHUNK)])
    pltpu.sync_copy(ly.at[pl.ds(base_e, _CHUNK)], ly_v.at[pl.ds(0, _CHUNK)])
    pltpu.sync_copy(lx.at[pl.ds(base_e, _CHUNK)], lx_v.at[pl.ds(0, _CHUNK)])
    pltpu.sync_copy(xo.at[pl.ds(base_e, _CHUNK)], xo_v.at[pl.ds(0, _CHUNK)])
    pltpu.sync_copy(yo.at[pl.ds(base_e, _CHUNK)], yo_v.at[pl.ds(0, _CHUNK)])
    pltpu.sync_copy(zo.at[pl.ds(base_e, _CHUNK)], zo_v.at[pl.ds(0, _CHUNK)])
    pltpu.sync_copy(it.at[pl.ds(base_e, _CHUNK)], it_v.at[pl.ds(0, _CHUNK)])

    ii = lax.iota(jnp.int32, _L)
    zvec = jnp.zeros((_L,), jnp.float32)

    def zb(j, _):
        zero_v[pl.ds(j * _L, _L)] = zvec
        return 0
    lax.fori_loop(0, _ZERO_W // _L, zb, 0)

    dump_idx = _DUMP + ii

    def round_body(r, _):
        z0 = r * _ROUND_Z

        def zr(j, _):
            pltpu.sync_copy(
                zero_v, acc.at[pl.ds(sid * _TEC_WB + j * _ZERO_W, _ZERO_W)])
            return 0
        lax.fori_loop(0, _TEC_WB // _ZERO_W, zr, 0)
        plsc.subcore_barrier()

        def emitter_body(i, rowcnt):
            elz = sload(lz_v, i)
            elb = sload(lb_v, i)
            zlo = jnp.maximum(z0, elz - (_SZ // 2))
            zhi = jnp.minimum(z0 + _ROUND_Z - 1, elz + (_SZ // 2))
            zub = jnp.where(elb == cid, zhi + 1, zlo)

            def z_body(zz, rowcnt):
                elc = sload(lc_v, i)
                ely = sload(ly_v, i)
                elx = sload(lx_v, i)
                dz = sload(zo_v, i) - 0.5
                dy = sload(yo_v, i) - 0.5
                dx = sload(xo_v, i) - 0.5
                fzi = jnp.where(dz < 0.0, -1, 0)
                fyi = jnp.where(dy < 0.0, -1, 0)
                fxi = jnp.where(dx < 0.0, -1, 0)
                wz1 = dz - fzi.astype(jnp.float32)
                wy1 = dy - fyi.astype(jnp.float32)
                wx1 = dx - fxi.astype(jnp.float32)
                wz0 = 1.0 - wz1
                wy0 = 1.0 - wy1
                wx0 = 1.0 - wx1
                cfv = cf_v[...]
                fac = sload(it_v, i) * _SCALE * jnp.where(elc == 0, cfv[0], cfv[1])
                wzf0 = wz0 * fac
                wzf1 = wz1 * fac

                k = zz - elz + (_SZ // 2)
                jz0 = k + 1 + fzi
                jz1 = jz0 + 1
                x0 = 1 + fxi
                x1 = x0 + 1
                sbase_z = (elc * _ROUND_Z + (zz - z0)) * _HW
                xbase = elx - (_SX // 2)
                xv = xbase + ii
                xok = (xv >= 0) & (xv < _W) & (ii < _SX)

                dylo = jnp.maximum(0, (_SY // 2) - ely)
                dyhi = jnp.minimum(_SY - 1, (_H - 1) - ely + (_SY // 2))

                bv = []
                for t in range(_SY + 1):
                    jy = t + 1 + fyi
                    xb0 = (pad_v[jz0, jy, pl.ds(x0, _L)] * wx0 +
                           pad_v[jz0, jy, pl.ds(x1, _L)] * wx1)
                    xb1 = (pad_v[jz1, jy, pl.ds(x0, _L)] * wx0 +
                           pad_v[jz1, jy, pl.ds(x1, _L)] * wx1)
                    bv.append(xb0 * wzf0 + xb1 * wzf1)

                y0 = ely - (_SY // 2)
                idx0 = jnp.where(xok, sbase_z + y0 * _W + xv, dump_idx)
                base_pos = rowcnt * _L
                for t in range(_SY):
                    row = bv[t] * wy0 + bv[t + 1] * wy1
                    yok = (t >= dylo) & (t <= dyhi)
                    ridx = jnp.where(yok, idx0 + t * _W, dump_idx)
                    val_v[pl.ds(base_pos + t * _L, _L)] = row
                    idx_v[pl.ds(base_pos + t * _L, _L)] = ridx
                rowcnt = rowcnt + _SY

                @pl.when(rowcnt == _NBUF_ROWS)
                def _():
                    pltpu.sync_copy(val_v, acc.at[idx_v], add=True)

                return jnp.where(rowcnt == _NBUF_ROWS, 0, rowcnt)

            return lax.fori_loop(zlo, zub, z_body, rowcnt)

        rowcnt = lax.fori_loop(0, _CHUNK, emitter_body, 0)

        def pad_body(j, _):
            pos = j * _L
            val_v[pl.ds(pos, _L)] = zvec
            idx_v[pl.ds(pos, _L)] = dump_idx
            return 0
        lax.fori_loop(rowcnt, _NBUF_ROWS, pad_body, 0)
        pltpu.sync_copy(val_v, acc.at[idx_v], add=True)

        plsc.subcore_barrier()

        bcl = sid // 8
        zof = (sid // 4) % 2
        yq = sid % 4
        hbm_off = ((2 * cid + bcl) * _DHW + (z0 + zof) * _HW
                   + yq * (_H // 4) * _W)
        pltpu.sync_copy(acc.at[pl.ds(sid * _TEC_WB, _TEC_WB)],
                        out.at[pl.ds(hbm_off, _TEC_WB)])
        return 0

    lax.fori_loop(0, _NROUNDS, round_body, 0)


@jax.jit
def _sc_place(lb, lc, lz, ly, lx, xo, yo, zo, it, pad, cf):
    mesh = plsc.VectorSubcoreMesh(core_axis_name="c", subcore_axis_name="s",
                                  num_cores=_NC, num_subcores=_NS)
    f = pl.kernel(
        _body,
        out_type=jax.ShapeDtypeStruct((_BS * _C * _D * _H * _W,), jnp.float32),
        mesh=mesh,
        scratch_types=[
            pltpu.VMEM_SHARED((_SLAB_WORDS + _ACC_EXTRA,), jnp.float32),
            pltpu.VMEM((_SZ + 2, _SY + 2, 24), jnp.float32),
            pltpu.VMEM((_CHUNK + _L,), jnp.int32),
            pltpu.VMEM((_CHUNK + _L,), jnp.int32),
            pltpu.VMEM((_CHUNK + _L,), jnp.int32),
            pltpu.VMEM((_CHUNK + _L,), jnp.int32),
            pltpu.VMEM((_CHUNK + _L,), jnp.int32),
            pltpu.VMEM((_CHUNK + _L,), jnp.float32),
            pltpu.VMEM((_CHUNK + _L,), jnp.float32),
            pltpu.VMEM((_CHUNK + _L,), jnp.float32),
            pltpu.VMEM((_CHUNK + _L,), jnp.float32),
            pltpu.VMEM((_L,), jnp.float32),
            pltpu.VMEM((_ZERO_W,), jnp.float32),
            pltpu.VMEM((_NBUF_ROWS * _L,), jnp.float32),
            pltpu.VMEM((_NBUF_ROWS * _L,), jnp.int32),
        ],
    )
    return f(lb, lc, lz, ly, lx, xo, yo, zo, it, pad, cf)


def kernel(loc_b, loc_c, loc_z, loc_y, loc_x, x_os, y_os, z_os, ints,
           psf_volume, channel_facs):
    psfc = jnp.maximum(psf_volume.astype(jnp.float32), 0.0)
    pad = jnp.pad(psfc, ((1, 1), (1, 1), (1, 24 - _SX - 1)))
    cf = jnp.zeros((_L,), jnp.float32).at[:_C].set(
        channel_facs.astype(jnp.float32))
    out = _sc_place(
        loc_b.astype(jnp.int32), loc_c.astype(jnp.int32),
        loc_z.astype(jnp.int32), loc_y.astype(jnp.int32),
        loc_x.astype(jnp.int32),
        x_os.astype(jnp.float32), y_os.astype(jnp.float32),
        z_os.astype(jnp.float32), ints.astype(jnp.float32),
        pad, cf)
    return out.reshape(_BS, _C, _D, _H, _W)


# speedup vs baseline: 1833.4189x; 1.1763x over previous
import functools

import jax
import jax.numpy as jnp
from jax import lax
from jax.experimental import pallas as pl
from jax.experimental.pallas import tpu as pltpu
from jax.experimental.pallas import tpu_sc as plsc

_N = 8192
_BS, _C, _D, _H, _W = 2, 2, 32, 512, 512
_SZ, _SY, _SX = 7, 15, 15
_SCALE = 10000.0

_NC = 2
_NS = 16
_L = 16

_ROUND_Z = 2
_NROUNDS = _D // _ROUND_Z
_SLAB_WORDS = _C * _ROUND_Z * _H * _W
_TEC_WB = _SLAB_WORDS // _NS
_DHW = _D * _H * _W
_HW = _H * _W
_CHUNK = _N // _NS
_NBUF_ROWS = 60
_DUMP = _SLAB_WORDS
_ZERO_W = 16384
_ACC_EXTRA = 7424


def _body(lb, lc, lz, ly, lx, xo, yo, zo, it, pad, cf,
          out,
          acc,
          pad_v, lb_v, lc_v, lz_v, ly_v, lx_v,
          xo_v, yo_v, zo_v, it_v, cf_v,
          zero_v, val_a, idx_a, val_b, idx_b, sem):
    cid = lax.axis_index("c")
    sid = lax.axis_index("s")
    base_e = sid * _CHUNK

    def sload(ref, i):
        return ref[pl.ds(i, _L)][0]

    pltpu.sync_copy(pad, pad_v)
    pltpu.sync_copy(cf, cf_v)
    pltpu.sync_copy(lb.at[pl.ds(base_e, _CHUNK)], lb_v.at[pl.ds(0, _CHUNK)])
    pltpu.sync_copy(lc.at[pl.ds(base_e, _CHUNK)], lc_v.at[pl.ds(0, _CHUNK)])
    pltpu.sync_copy(lz.at[pl.ds(base_e, _CHUNK)], lz_v.at[pl.ds(0, _CHUNK)])
    pltpu.sync_copy(ly.at[pl.ds(base_e, _CHUNK)], ly_v.at[pl.ds(0, _CHUNK)])
    pltpu.sync_copy(lx.at[pl.ds(base_e, _CHUNK)], lx_v.at[pl.ds(0, _CHUNK)])
    pltpu.sync_copy(xo.at[pl.ds(base_e, _CHUNK)], xo_v.at[pl.ds(0, _CHUNK)])
    pltpu.sync_copy(yo.at[pl.ds(base_e, _CHUNK)], yo_v.at[pl.ds(0, _CHUNK)])
    pltpu.sync_copy(zo.at[pl.ds(base_e, _CHUNK)], zo_v.at[pl.ds(0, _CHUNK)])
    pltpu.sync_copy(it.at[pl.ds(base_e, _CHUNK)], it_v.at[pl.ds(0, _CHUNK)])

    ii = lax.iota(jnp.int32, _L)
    zvec = jnp.zeros((_L,), jnp.float32)

    def zb(j, _):
        zero_v[pl.ds(j * _L, _L)] = zvec
        return 0
    lax.fori_loop(0, _ZERO_W // _L, zb, 0)

    dump_idx = _DUMP + ii

    def round_body(r, _):
        z0 = r * _ROUND_Z

        def zr(j, _):
            pltpu.sync_copy(
                zero_v, acc.at[pl.ds(sid * _TEC_WB + j * _ZERO_W, _ZERO_W)])
            return 0
        lax.fori_loop(0, _TEC_WB // _ZERO_W, zr, 0)
        plsc.subcore_barrier()

        def emitter_body(i, carry):
            elz = sload(lz_v, i)
            elb = sload(lb_v, i)
            zlo = jnp.maximum(z0, elz - (_SZ // 2))
            zhi = jnp.minimum(z0 + _ROUND_Z - 1, elz + (_SZ // 2))
            zub = jnp.where(elb == cid, zhi + 1, zlo)

            def z_body(zz, carry):
                rowcnt, fcnt = carry
                elc = sload(lc_v, i)
                ely = sload(ly_v, i)
                elx = sload(lx_v, i)
                dz = sload(zo_v, i) - 0.5
                dy = sload(yo_v, i) - 0.5
                dx = sload(xo_v, i) - 0.5
                fzi = jnp.where(dz < 0.0, -1, 0)
                fyi = jnp.where(dy < 0.0, -1, 0)
                fxi = jnp.where(dx < 0.0, -1, 0)
                wz1 = dz - fzi.astype(jnp.float32)
                wy1 = dy - fyi.astype(jnp.float32)
                wx1 = dx - fxi.astype(jnp.float32)
                wz0 = 1.0 - wz1
                wy0 = 1.0 - wy1
                wx0 = 1.0 - wx1
                cfv = cf_v[...]
                fac = sload(it_v, i) * _SCALE * jnp.where(elc == 0, cfv[0], cfv[1])
                wzf0 = wz0 * fac
                wzf1 = wz1 * fac

                k = zz - elz + (_SZ // 2)
                jz0 = k + 1 + fzi
                jz1 = jz0 + 1
                x0 = 1 + fxi
                x1 = x0 + 1
                sbase_z = (elc * _ROUND_Z + (zz - z0)) * _HW
                xbase = elx - (_SX // 2)
                xv = xbase + ii
                xok = (xv >= 0) & (xv < _W) & (ii < _SX)

                dylo = jnp.maximum(0, (_SY // 2) - ely)
                dyhi = jnp.minimum(_SY - 1, (_H - 1) - ely + (_SY // 2))

                bv = []
                for t in range(_SY + 1):
                    jy = t + 1 + fyi
                    xb0 = (pad_v[jz0, jy, pl.ds(x0, _L)] * wx0 +
                           pad_v[jz0, jy, pl.ds(x1, _L)] * wx1)
                    xb1 = (pad_v[jz1, jy, pl.ds(x0, _L)] * wx0 +
                           pad_v[jz1, jy, pl.ds(x1, _L)] * wx1)
                    bv.append(xb0 * wzf0 + xb1 * wzf1)

                y0 = ely - (_SY // 2)
                idx0 = jnp.where(xok, sbase_z + y0 * _W + xv, dump_idx)
                p = fcnt & 1
                base_pos = rowcnt * _L
                rows = []
                for t in range(_SY):
                    row = bv[t] * wy0 + bv[t + 1] * wy1
                    yok = (t >= dylo) & (t <= dyhi)
                    ridx = jnp.where(yok, idx0 + t * _W, dump_idx)
                    rows.append((row, ridx))

                def stage(val_r, idx_r):
                    for t, (row, ridx) in enumerate(rows):
                        val_r[pl.ds(base_pos + t * _L, _L)] = row
                        idx_r[pl.ds(base_pos + t * _L, _L)] = ridx

                @pl.when(p == 0)
                def _():
                    stage(val_a, idx_a)

                @pl.when(p == 1)
                def _():
                    stage(val_b, idx_b)

                rowcnt = rowcnt + _SY
                full = rowcnt == _NBUF_ROWS

                @pl.when(full & (p == 0))
                def _():
                    pltpu.async_copy(val_a, acc.at[idx_a], sem.at[0],
                                     add=True)

                    @pl.when(fcnt >= 1)
                    def _():
                        pltpu.make_async_copy(val_b, acc.at[idx_b],
                                              sem.at[1]).wait()

                @pl.when(full & (p == 1))
                def _():
                    pltpu.async_copy(val_b, acc.at[idx_b], sem.at[1],
                                     add=True)
                    pltpu.make_async_copy(val_a, acc.at[idx_a],
                                          sem.at[0]).wait()
                return (jnp.where(full, 0, rowcnt),
                        jnp.where(full, fcnt + 1, fcnt))

            return lax.fori_loop(zlo, zub, z_body, carry)

        rowcnt, fcnt = lax.fori_loop(0, _CHUNK, emitter_body, (0, 0))

        pf = fcnt & 1

        @pl.when(pf == 0)
        def _():
            def pad_body(j, _):
                pos = j * _L
                val_a[pl.ds(pos, _L)] = zvec
                idx_a[pl.ds(pos, _L)] = dump_idx
                return 0
            lax.fori_loop(rowcnt, _NBUF_ROWS, pad_body, 0)
            pltpu.sync_copy(val_a, acc.at[idx_a], add=True)

            @pl.when(fcnt >= 1)
            def _():
                pltpu.make_async_copy(val_b, acc.at[idx_b], sem.at[1]).wait()

        @pl.when(pf == 1)
        def _():
            def pad_body(j, _):
                pos = j * _L
                val_b[pl.ds(pos, _L)] = zvec
                idx_b[pl.ds(pos, _L)] = dump_idx
                return 0
            lax.fori_loop(rowcnt, _NBUF_ROWS, pad_body, 0)
            pltpu.sync_copy(val_b, acc.at[idx_b], add=True)
            pltpu.make_async_copy(val_a, acc.at[idx_a], sem.at[0]).wait()

        plsc.subcore_barrier()

        bcl = sid // 8
        zof = (sid // 4) % 2
        yq = sid % 4
        hbm_off = ((2 * cid + bcl) * _DHW + (z0 + zof) * _HW
                   + yq * (_H // 4) * _W)
        pltpu.sync_copy(acc.at[pl.ds(sid * _TEC_WB, _TEC_WB)],
                        out.at[pl.ds(hbm_off, _TEC_WB)])
        return 0

    lax.fori_loop(0, _NROUNDS, round_body, 0)


@jax.jit
def _sc_place(lb, lc, lz, ly, lx, xo, yo, zo, it, pad, cf):
    mesh = plsc.VectorSubcoreMesh(core_axis_name="c", subcore_axis_name="s",
                                  num_cores=_NC, num_subcores=_NS)
    f = pl.kernel(
        _body,
        out_type=jax.ShapeDtypeStruct((_BS * _C * _D * _H * _W,), jnp.float32),
        mesh=mesh,
        scratch_types=[
            pltpu.VMEM_SHARED((_SLAB_WORDS + _ACC_EXTRA,), jnp.float32),
            pltpu.VMEM((_SZ + 2, _SY + 2, 24), jnp.float32),
            pltpu.VMEM((_CHUNK + _L,), jnp.int32),
            pltpu.VMEM((_CHUNK + _L,), jnp.int32),
            pltpu.VMEM((_CHUNK + _L,), jnp.int32),
            pltpu.VMEM((_CHUNK + _L,), jnp.int32),
            pltpu.VMEM((_CHUNK + _L,), jnp.int32),
            pltpu.VMEM((_CHUNK + _L,), jnp.float32),
            pltpu.VMEM((_CHUNK + _L,), jnp.float32),
            pltpu.VMEM((_CHUNK + _L,), jnp.float32),
            pltpu.VMEM((_CHUNK + _L,), jnp.float32),
            pltpu.VMEM((_L,), jnp.float32),
            pltpu.VMEM((_ZERO_W,), jnp.float32),
            pltpu.VMEM((_NBUF_ROWS * _L,), jnp.float32),
            pltpu.VMEM((_NBUF_ROWS * _L,), jnp.int32),
            pltpu.VMEM((_NBUF_ROWS * _L,), jnp.float32),
            pltpu.VMEM((_NBUF_ROWS * _L,), jnp.int32),
            pltpu.SemaphoreType.DMA((2,)),
        ],
    )
    return f(lb, lc, lz, ly, lx, xo, yo, zo, it, pad, cf)


def kernel(loc_b, loc_c, loc_z, loc_y, loc_x, x_os, y_os, z_os, ints,
           psf_volume, channel_facs):
    psfc = jnp.maximum(psf_volume.astype(jnp.float32), 0.0)
    pad = jnp.pad(psfc, ((1, 1), (1, 1), (1, 24 - _SX - 1)))
    cf = jnp.zeros((_L,), jnp.float32).at[:_C].set(
        channel_facs.astype(jnp.float32))
    out = _sc_place(
        loc_b.astype(jnp.int32), loc_c.astype(jnp.int32),
        loc_z.astype(jnp.int32), loc_y.astype(jnp.int32),
        loc_x.astype(jnp.int32),
        x_os.astype(jnp.float32), y_os.astype(jnp.float32),
        z_os.astype(jnp.float32), ints.astype(jnp.float32),
        pad, cf)
    return out.reshape(_BS, _C, _D, _H, _W)
